# baseline (speedup 1.0000x reference)
import functools

import jax
import jax.numpy as jnp
from jax import lax
from jax.experimental import pallas as pl
from jax.experimental.pallas import tpu as pltpu
from jax.experimental.pallas import tpu_sc as plsc

N = 320000
D = 128
H = 128
NUM_SEG = 2048

NW = 32
SEGC = NUM_SEG // 16

BR = 12800
NBLK = N // BR
NBLK_A = 16
NBLK_B = NBLK - NBLK_A
ROWS_A = NBLK_A * BR
ROWS_B = NBLK_B * BR


def _silu(v):
    return v * (0.5 * jnp.tanh(0.5 * v) + 0.5)


def _mlp_body(x_ref, w1_ref, b1_ref, w2t_ref, b2_ref, w3t_ref, b3_ref, o_ref):
    x = x_ref[...]
    h = jnp.dot(x, w1_ref[...], preferred_element_type=jnp.float32) + b1_ref[...]
    h = _silu(h)
    ht = h.T
    g = jnp.dot(w2t_ref[...], ht, preferred_element_type=jnp.float32) + b2_ref[...]
    g = _silu(g)
    o = jnp.dot(w3t_ref[...], g, preferred_element_type=jnp.float32) + b3_ref[...]
    i = pl.program_id(0)
    o_ref[pl.ds(i * BR, BR)] = o.reshape(BR)


def _mlp_part(off_blocks, nblk):
    rows = nblk * BR

    def call(x, W1, b1r, W2t, b2c, w3t, b3r):
        return pl.pallas_call(
            _mlp_body,
            grid=(nblk,),
            in_specs=[
                pl.BlockSpec((BR, D), lambda i: (i + off_blocks, 0)),
                pl.BlockSpec((D, H), lambda i: (0, 0)),
                pl.BlockSpec((1, H), lambda i: (0, 0)),
                pl.BlockSpec((H, H), lambda i: (0, 0)),
                pl.BlockSpec((H, 1), lambda i: (0, 0)),
                pl.BlockSpec((1, H), lambda i: (0, 0)),
                pl.BlockSpec((1, 1), lambda i: (0, 0)),
            ],
            out_specs=pl.BlockSpec((rows,), lambda i: (0,)),
            out_shape=jax.ShapeDtypeStruct((rows,), jnp.float32),
        )(x, W1, b1r, W2t, b2c, w3t, b3r)

    return call


_mlp_a = _mlp_part(0, NBLK_A)
_mlp_b = _mlp_part(NBLK_A, NBLK_B)


def _shuf(vec, idx):
    return jnp.take_along_axis(vec, idx, axis=0, mode="promise_in_bounds")


def _make_sc(start, rows):
    rw = rows // NW
    ch = rw // 16

    @functools.partial(
        pl.kernel,
        out_type=jax.ShapeDtypeStruct((2 * NUM_SEG,), jnp.float32),
        mesh=plsc.VectorSubcoreMesh(core_axis_name="c", subcore_axis_name="s"),
        scratch_types=[
            pltpu.VMEM((rw,), jnp.float32),
            pltpu.VMEM((rw,), jnp.int32),
            pltpu.VMEM((rw,), jnp.int32),
            pltpu.VMEM((16,), jnp.float32),
            pltpu.VMEM((NUM_SEG,), jnp.float32),
            pltpu.VMEM((NUM_SEG,), jnp.float32),
            pltpu.VMEM((NUM_SEG,), jnp.float32),
            pltpu.VMEM((128,), jnp.float32),
            pltpu.VMEM((128,), jnp.float32),
            pltpu.VMEM_SHARED((16, NUM_SEG), jnp.float32),
        ],
        compiler_params=pltpu.CompilerParams(needs_layout_passes=False),
    )
    def _sc_segsum(y_hbm, z_hbm, b_hbm, tab_hbm, out_hbm,
                   vals_v, z_v, b_v, tab_v, start_v, end_v, diff_v,
                   red_v, tmp_v, acc_sh):
        c = lax.axis_index("c")
        s = lax.axis_index("s")
        w = s * 2 + c
        base = start + w * rw

        pltpu.sync_copy(y_hbm.at[pl.ds(w * rw, rw)], vals_v)
        pltpu.sync_copy(z_hbm.at[pl.ds(base, rw)], z_v)
        pltpu.sync_copy(b_hbm.at[pl.ds(base, rw)], b_v)
        pltpu.sync_copy(tab_hbm, tab_v)

        zero16 = jnp.zeros((16,), jnp.float32)

        def zbody(i, carry):
            start_v[pl.ds(i * 16, 16)] = zero16
            end_v[pl.ds(i * 16, 16)] = zero16
            return carry

        lax.fori_loop(0, SEGC, zbody, 0)

        iota = lax.iota(jnp.int32, 16)
        prev_idx = jnp.maximum(iota - 1, 0)
        next_idx = jnp.minimum(iota + 1, 15)
        last_idx = jnp.full((16,), 15, jnp.int32)
        m0 = iota == 0
        m15 = iota == 15

        def body(i, carry):
            run_vec, prevb_vec = carry
            off = i * 16
            b = b_v[pl.ds(off, 16)]
            v = vals_v[pl.ds(off, 16)]
            zc = z_v[pl.ds(off, 16)]
            v = v + plsc.load_gather(tab_v, [zc])
            inc = plsc.cumsum(v) + run_vec
            prevv = jnp.where(m0, prevb_vec, _shuf(b, prev_idx))
            nextv = _shuf(b, next_idx)
            startm = b != prevv
            endm = (b != nextv) | m15
            plsc.store_scatter(start_v, [b], inc - v, mask=startm)
            plsc.store_scatter(end_v, [b], inc, mask=endm)
            return (_shuf(inc, last_idx), _shuf(b, last_idx))

        lax.fori_loop(0, ch, body,
                      (jnp.zeros((16,), jnp.float32),
                       jnp.full((16,), -1, jnp.int32)))

        def dbody(i, carry):
            sl = pl.ds(i * 16, 16)
            diff_v[sl] = end_v[sl] - start_v[sl]
            return carry

        lax.fori_loop(0, SEGC, dbody, 0)

        pltpu.sync_copy(diff_v, acc_sh.at[s])
        plsc.subcore_barrier()

        for k in range(8):
            red_v[pl.ds(k * 16, 16)] = zero16
        for j in range(16):
            pltpu.sync_copy(acc_sh.at[j, pl.ds(s * 128, 128)], tmp_v)
            for k in range(8):
                sl = pl.ds(k * 16, 16)
                red_v[sl] = red_v[sl] + tmp_v[sl]
        pltpu.sync_copy(red_v, out_hbm.at[pl.ds(c * NUM_SEG + s * 128, 128)])

    return _sc_segsum


_sc_a = _make_sc(0, ROWS_A)
_sc_b = _make_sc(ROWS_A, ROWS_B)


def _comb_body(p_ref, o_ref):
    o_ref[...] = jnp.sum(p_ref[...], axis=0, keepdims=True)


def _combine(partials):
    return pl.pallas_call(
        _comb_body,
        out_shape=jax.ShapeDtypeStruct((1, NUM_SEG), jnp.float32),
    )(partials)


def kernel(x, Z, batch, W1, b1, W2, b2, W3, b3, atom_ref):
    b1r = b1.reshape(1, H)
    b2c = b2.reshape(H, 1)
    w2t = W2.T
    w3t = W3.reshape(1, H)
    b3r = b3.reshape(1, 1)
    z32 = Z.astype(jnp.int32)
    b32 = batch.astype(jnp.int32)
    tab = jnp.pad(atom_ref.reshape(-1), (0, 16 - atom_ref.shape[0]))

    y_a = _mlp_a(x, W1, b1r, w2t, b2c, w3t, b3r)
    y_b = _mlp_b(x, W1, b1r, w2t, b2c, w3t, b3r)
    p_a = _sc_a(y_a, z32, b32, tab)
    p_b = _sc_b(y_b, z32, b32, tab)
    partials = jnp.concatenate(
        [p_a.reshape(2, NUM_SEG), p_b.reshape(2, NUM_SEG)], axis=0)
    out = _combine(partials)
    return out.reshape(NUM_SEG, 1)


# speedup vs baseline: 1.0452x; 1.0452x over previous
import functools

import jax
import jax.numpy as jnp
from jax import lax
from jax.experimental import pallas as pl
from jax.experimental.pallas import tpu as pltpu
from jax.experimental.pallas import tpu_sc as plsc

N = 320000
D = 128
H = 128
NUM_SEG = 2048

NW = 32
SEGC = NUM_SEG // 16

BR = 16000
NBLK = N // BR
NBLK_A = 12
NBLK_B = NBLK - NBLK_A
ROWS_A = NBLK_A * BR
ROWS_B = NBLK_B * BR


def _silu(v):
    return v * (0.5 * jnp.tanh(0.5 * v) + 0.5)


def _mlp_body(x_ref, w1_ref, b1_ref, w2t_ref, b2_ref, w3t_ref, b3_ref, o_ref):
    x = x_ref[...]
    h = jnp.dot(x, w1_ref[...], preferred_element_type=jnp.float32) + b1_ref[...]
    h = _silu(h)
    ht = h.T
    g = jnp.dot(w2t_ref[...], ht, preferred_element_type=jnp.float32) + b2_ref[...]
    g = _silu(g)
    o = jnp.dot(w3t_ref[...], g, preferred_element_type=jnp.float32) + b3_ref[...]
    i = pl.program_id(0)
    o_ref[pl.ds(i * BR, BR)] = o.reshape(BR)


def _mlp_part(off_blocks, nblk):
    rows = nblk * BR

    def call(x, W1, b1r, W2t, b2c, w3t, b3r):
        return pl.pallas_call(
            _mlp_body,
            grid=(nblk,),
            in_specs=[
                pl.BlockSpec((BR, D), lambda i: (i + off_blocks, 0)),
                pl.BlockSpec((D, H), lambda i: (0, 0)),
                pl.BlockSpec((1, H), lambda i: (0, 0)),
                pl.BlockSpec((H, H), lambda i: (0, 0)),
                pl.BlockSpec((H, 1), lambda i: (0, 0)),
                pl.BlockSpec((1, H), lambda i: (0, 0)),
                pl.BlockSpec((1, 1), lambda i: (0, 0)),
            ],
            out_specs=pl.BlockSpec((rows,), lambda i: (0,)),
            out_shape=jax.ShapeDtypeStruct((rows,), jnp.float32),
        )(x, W1, b1r, W2t, b2c, w3t, b3r)

    return call


_mlp_a = _mlp_part(0, NBLK_A)
_mlp_b = _mlp_part(NBLK_A, NBLK_B)


def _shuf(vec, idx):
    return jnp.take_along_axis(vec, idx, axis=0, mode="promise_in_bounds")


def _make_sc(start, rows):
    rw = rows // NW
    ch = rw // 16

    @functools.partial(
        pl.kernel,
        out_type=jax.ShapeDtypeStruct((2 * NUM_SEG,), jnp.float32),
        mesh=plsc.VectorSubcoreMesh(core_axis_name="c", subcore_axis_name="s"),
        scratch_types=[
            pltpu.VMEM((rw,), jnp.float32),
            pltpu.VMEM((rw,), jnp.int32),
            pltpu.VMEM((rw,), jnp.int32),
            pltpu.VMEM((16,), jnp.float32),
            pltpu.VMEM((NUM_SEG,), jnp.float32),
            pltpu.VMEM((NUM_SEG,), jnp.float32),
            pltpu.VMEM((NUM_SEG,), jnp.float32),
            pltpu.VMEM((128,), jnp.float32),
            pltpu.VMEM((128,), jnp.float32),
            pltpu.VMEM_SHARED((16, NUM_SEG), jnp.float32),
        ],
        compiler_params=pltpu.CompilerParams(needs_layout_passes=False),
    )
    def _sc_segsum(y_hbm, z_hbm, b_hbm, tab_hbm, out_hbm,
                   vals_v, z_v, b_v, tab_v, start_v, end_v, diff_v,
                   red_v, tmp_v, acc_sh):
        c = lax.axis_index("c")
        s = lax.axis_index("s")
        w = s * 2 + c
        base = start + w * rw

        pltpu.sync_copy(y_hbm.at[pl.ds(w * rw, rw)], vals_v)
        pltpu.sync_copy(z_hbm.at[pl.ds(base, rw)], z_v)
        pltpu.sync_copy(b_hbm.at[pl.ds(base, rw)], b_v)
        pltpu.sync_copy(tab_hbm, tab_v)

        zero16 = jnp.zeros((16,), jnp.float32)

        def zbody(i, carry):
            start_v[pl.ds(i * 16, 16)] = zero16
            end_v[pl.ds(i * 16, 16)] = zero16
            return carry

        lax.fori_loop(0, SEGC, zbody, 0)

        iota = lax.iota(jnp.int32, 16)
        prev_idx = jnp.maximum(iota - 1, 0)
        next_idx = jnp.minimum(iota + 1, 15)
        last_idx = jnp.full((16,), 15, jnp.int32)
        m0 = iota == 0
        m15 = iota == 15

        def body(i, carry):
            run_vec, prevb_vec = carry
            off = i * 16
            b = b_v[pl.ds(off, 16)]
            v = vals_v[pl.ds(off, 16)]
            zc = z_v[pl.ds(off, 16)]
            v = v + plsc.load_gather(tab_v, [zc])
            inc = plsc.cumsum(v) + run_vec
            prevv = jnp.where(m0, prevb_vec, _shuf(b, prev_idx))
            nextv = _shuf(b, next_idx)
            startm = b != prevv
            endm = (b != nextv) | m15
            plsc.store_scatter(start_v, [b], inc - v, mask=startm)
            plsc.store_scatter(end_v, [b], inc, mask=endm)
            return (_shuf(inc, last_idx), _shuf(b, last_idx))

        lax.fori_loop(0, ch, body,
                      (jnp.zeros((16,), jnp.float32),
                       jnp.full((16,), -1, jnp.int32)))

        def dbody(i, carry):
            sl = pl.ds(i * 16, 16)
            diff_v[sl] = end_v[sl] - start_v[sl]
            return carry

        lax.fori_loop(0, SEGC, dbody, 0)

        pltpu.sync_copy(diff_v, acc_sh.at[s])
        plsc.subcore_barrier()

        for k in range(8):
            red_v[pl.ds(k * 16, 16)] = zero16
        for j in range(16):
            pltpu.sync_copy(acc_sh.at[j, pl.ds(s * 128, 128)], tmp_v)
            for k in range(8):
                sl = pl.ds(k * 16, 16)
                red_v[sl] = red_v[sl] + tmp_v[sl]
        pltpu.sync_copy(red_v, out_hbm.at[pl.ds(c * NUM_SEG + s * 128, 128)])

    return _sc_segsum


_sc_a = _make_sc(0, ROWS_A)
_sc_b = _make_sc(ROWS_A, ROWS_B)


def _comb_body(p_ref, o_ref):
    o_ref[...] = jnp.sum(p_ref[...], axis=0, keepdims=True)


def _combine(partials):
    return pl.pallas_call(
        _comb_body,
        out_shape=jax.ShapeDtypeStruct((1, NUM_SEG), jnp.float32),
    )(partials)


def kernel(x, Z, batch, W1, b1, W2, b2, W3, b3, atom_ref):
    b1r = b1.reshape(1, H)
    b2c = b2.reshape(H, 1)
    w2t = W2.T
    w3t = W3.reshape(1, H)
    b3r = b3.reshape(1, 1)
    z32 = Z.astype(jnp.int32)
    b32 = batch.astype(jnp.int32)
    tab = jnp.pad(atom_ref.reshape(-1), (0, 16 - atom_ref.shape[0]))

    y_a = _mlp_a(x, W1, b1r, w2t, b2c, w3t, b3r)
    y_b = _mlp_b(x, W1, b1r, w2t, b2c, w3t, b3r)
    p_a = _sc_a(y_a, z32, b32, tab)
    p_b = _sc_b(y_b, z32, b32, tab)
    partials = jnp.concatenate(
        [p_a.reshape(2, NUM_SEG), p_b.reshape(2, NUM_SEG)], axis=0)
    out = _combine(partials)
    return out.reshape(NUM_SEG, 1)


# speedup vs baseline: 1.0762x; 1.0296x over previous
import functools

import jax
import jax.numpy as jnp
from jax import lax
from jax.experimental import pallas as pl
from jax.experimental.pallas import tpu as pltpu
from jax.experimental.pallas import tpu_sc as plsc

N = 320000
D = 128
H = 128
NUM_SEG = 2048

NW = 32
SEGC = NUM_SEG // 16

BR = 16000
NBLK = N // BR
NBLK_A = 12
NBLK_B = NBLK - NBLK_A
ROWS_A = NBLK_A * BR
ROWS_B = NBLK_B * BR


def _silu(v):
    return v * (0.5 * jnp.tanh(0.5 * v) + 0.5)


def _mlp_body(x_ref, w1_ref, b1_ref, w2t_ref, b2_ref, w3t_ref, b3_ref, o_ref):
    x = x_ref[...]
    h = jnp.dot(x, w1_ref[...], preferred_element_type=jnp.float32) + b1_ref[...]
    h = _silu(h)
    ht = h.T
    g = jnp.dot(w2t_ref[...], ht, preferred_element_type=jnp.float32) + b2_ref[...]
    g = _silu(g)
    o = jnp.dot(w3t_ref[...], g, preferred_element_type=jnp.float32) + b3_ref[...]
    i = pl.program_id(0)
    o_ref[pl.ds(i * BR, BR)] = o.reshape(BR)


def _mlp_part(off_blocks, nblk):
    rows = nblk * BR

    def call(x, W1, b1r, W2t, b2c, w3t, b3r):
        return pl.pallas_call(
            _mlp_body,
            grid=(nblk,),
            in_specs=[
                pl.BlockSpec((BR, D), lambda i: (i + off_blocks, 0)),
                pl.BlockSpec((D, H), lambda i: (0, 0)),
                pl.BlockSpec((1, H), lambda i: (0, 0)),
                pl.BlockSpec((H, H), lambda i: (0, 0)),
                pl.BlockSpec((H, 1), lambda i: (0, 0)),
                pl.BlockSpec((1, H), lambda i: (0, 0)),
                pl.BlockSpec((1, 1), lambda i: (0, 0)),
            ],
            out_specs=pl.BlockSpec((rows,), lambda i: (0,)),
            out_shape=jax.ShapeDtypeStruct((rows,), jnp.float32),
        )(x, W1, b1r, W2t, b2c, w3t, b3r)

    return call


_mlp_a = _mlp_part(0, NBLK_A)
_mlp_b = _mlp_part(NBLK_A, NBLK_B)


def _shuf(vec, idx):
    return jnp.take_along_axis(vec, idx, axis=0, mode="promise_in_bounds")


def _make_sc(start, rows):
    rw = rows // NW
    ch = rw // 16

    @functools.partial(
        pl.kernel,
        out_type=jax.ShapeDtypeStruct((2 * NUM_SEG,), jnp.float32),
        mesh=plsc.VectorSubcoreMesh(core_axis_name="c", subcore_axis_name="s"),
        scratch_types=[
            pltpu.VMEM((rw,), jnp.float32),
            pltpu.VMEM((rw,), jnp.int32),
            pltpu.VMEM((rw,), jnp.int32),
            pltpu.VMEM((16,), jnp.float32),
            pltpu.VMEM((NUM_SEG,), jnp.float32),
            pltpu.VMEM((NUM_SEG,), jnp.float32),
            pltpu.VMEM((NUM_SEG,), jnp.float32),
            pltpu.VMEM((128,), jnp.float32),
            pltpu.VMEM((16, 128), jnp.float32),
            pltpu.VMEM_SHARED((16, NUM_SEG), jnp.float32),
            pltpu.SemaphoreType.DMA,
        ],
        compiler_params=pltpu.CompilerParams(needs_layout_passes=False),
    )
    def _sc_segsum(y_hbm, z_hbm, b_hbm, tab_hbm, out_hbm,
                   vals_v, z_v, b_v, tab_v, start_v, end_v, diff_v,
                   red_v, tmp_v, acc_sh, sem):
        c = lax.axis_index("c")
        s = lax.axis_index("s")
        w = s * 2 + c
        base = start + w * rw

        cps = [
            pltpu.async_copy(y_hbm.at[pl.ds(w * rw, rw)], vals_v, sem),
            pltpu.async_copy(z_hbm.at[pl.ds(base, rw)], z_v, sem),
            pltpu.async_copy(b_hbm.at[pl.ds(base, rw)], b_v, sem),
            pltpu.async_copy(tab_hbm, tab_v, sem),
        ]

        zero16 = jnp.zeros((16,), jnp.float32)

        def zbody(i, carry):
            start_v[pl.ds(i * 16, 16)] = zero16
            end_v[pl.ds(i * 16, 16)] = zero16
            return carry

        lax.fori_loop(0, SEGC, zbody, 0)
        for cp in cps:
            cp.wait()

        iota = lax.iota(jnp.int32, 16)
        prev_idx = jnp.maximum(iota - 1, 0)
        next_idx = jnp.minimum(iota + 1, 15)
        last_idx = jnp.full((16,), 15, jnp.int32)
        m0 = iota == 0
        m15 = iota == 15

        def body(i, carry):
            run_vec, prevb_vec = carry
            off = i * 16
            b = b_v[pl.ds(off, 16)]
            v = vals_v[pl.ds(off, 16)]
            zc = z_v[pl.ds(off, 16)]
            v = v + plsc.load_gather(tab_v, [zc])
            inc = plsc.cumsum(v) + run_vec
            prevv = jnp.where(m0, prevb_vec, _shuf(b, prev_idx))
            nextv = _shuf(b, next_idx)
            startm = b != prevv
            endm = (b != nextv) | m15
            plsc.store_scatter(start_v, [b], inc - v, mask=startm)
            plsc.store_scatter(end_v, [b], inc, mask=endm)
            return (_shuf(inc, last_idx), _shuf(b, last_idx))

        lax.fori_loop(0, ch, body,
                      (jnp.zeros((16,), jnp.float32),
                       jnp.full((16,), -1, jnp.int32)))

        def dbody(i, carry):
            sl = pl.ds(i * 16, 16)
            diff_v[sl] = end_v[sl] - start_v[sl]
            return carry

        lax.fori_loop(0, SEGC, dbody, 0)

        pltpu.sync_copy(diff_v, acc_sh.at[s])
        plsc.subcore_barrier()

        rcps = [
            pltpu.async_copy(acc_sh.at[j, pl.ds(s * 128, 128)], tmp_v.at[j], sem)
            for j in range(16)
        ]
        for k in range(8):
            red_v[pl.ds(k * 16, 16)] = zero16
        for cp in rcps:
            cp.wait()
        for j in range(16):
            for k in range(8):
                sl = pl.ds(k * 16, 16)
                red_v[sl] = red_v[sl] + tmp_v[j, pl.ds(k * 16, 16)]
        pltpu.sync_copy(red_v, out_hbm.at[pl.ds(c * NUM_SEG + s * 128, 128)])

    return _sc_segsum


_sc_a = _make_sc(0, ROWS_A)
_sc_b = _make_sc(ROWS_A, ROWS_B)


def _comb_body(p_ref, o_ref):
    o_ref[...] = jnp.sum(p_ref[...], axis=0, keepdims=True)


def _combine(partials):
    return pl.pallas_call(
        _comb_body,
        out_shape=jax.ShapeDtypeStruct((1, NUM_SEG), jnp.float32),
    )(partials)


def kernel(x, Z, batch, W1, b1, W2, b2, W3, b3, atom_ref):
    b1r = b1.reshape(1, H)
    b2c = b2.reshape(H, 1)
    w2t = W2.T
    w3t = W3.reshape(1, H)
    b3r = b3.reshape(1, 1)
    z32 = Z.astype(jnp.int32)
    b32 = batch.astype(jnp.int32)
    tab = jnp.pad(atom_ref.reshape(-1), (0, 16 - atom_ref.shape[0]))

    y_a = _mlp_a(x, W1, b1r, w2t, b2c, w3t, b3r)
    y_b = _mlp_b(x, W1, b1r, w2t, b2c, w3t, b3r)
    p_a = _sc_a(y_a, z32, b32, tab)
    p_b = _sc_b(y_b, z32, b32, tab)
    partials = jnp.concatenate(
        [p_a.reshape(2, NUM_SEG), p_b.reshape(2, NUM_SEG)], axis=0)
    out = _combine(partials)
    return out.reshape(NUM_SEG, 1)
